# Initial kernel scaffold; baseline (speedup 1.0000x reference)
import jax
import jax.numpy as jnp
from jax.experimental import pallas as pl


def kernel(x, pos_table, gamma, beta):
    raise NotImplementedError("write your pallas kernel here")


# speedup vs baseline: 1.4183x; 1.4183x over previous
import functools

import jax
import jax.numpy as jnp
from jax.experimental import pallas as pl

_EPS = 1e-5
_SB = 256


def _stats_body(x_ref, pe_ref, sums_ref):
    i = pl.program_id(0)
    y = x_ref[...] + pe_ref[...][:, None, :]
    s = jnp.sum(y, axis=(0, 1))
    q = jnp.sum(y * y, axis=(0, 1))
    part = jnp.stack([s, q])

    @pl.when(i == 0)
    def _():
        sums_ref[...] = part

    @pl.when(i != 0)
    def _():
        sums_ref[...] += part


def _norm_body(sums_ref, gamma_ref, beta_ref, x_ref, pe_ref, o_ref, *, n):
    mean = sums_ref[0:1, :] / n
    var = sums_ref[1:2, :] / n - mean * mean
    inv = jax.lax.rsqrt(var + _EPS)
    g = gamma_ref[...] * inv
    off = beta_ref[...] - mean * g
    row = pe_ref[...] * g + off
    o_ref[...] = x_ref[...] * g[:, None, :] + row[:, None, :]


def kernel(x, pos_table, gamma, beta):
    S, B, D = x.shape
    n = float(S * B)
    grid = (S // _SB,)

    sums = pl.pallas_call(
        _stats_body,
        grid=grid,
        in_specs=[
            pl.BlockSpec((_SB, B, D), lambda i: (i, 0, 0)),
            pl.BlockSpec((_SB, D), lambda i: (i, 0)),
        ],
        out_specs=pl.BlockSpec((2, D), lambda i: (0, 0)),
        out_shape=jax.ShapeDtypeStruct((2, D), jnp.float32),
    )(x, pos_table[:S])

    out = pl.pallas_call(
        functools.partial(_norm_body, n=n),
        grid=grid,
        in_specs=[
            pl.BlockSpec((2, D), lambda i: (0, 0)),
            pl.BlockSpec((1, D), lambda i: (0, 0)),
            pl.BlockSpec((1, D), lambda i: (0, 0)),
            pl.BlockSpec((_SB, B, D), lambda i: (i, 0, 0)),
            pl.BlockSpec((_SB, D), lambda i: (i, 0)),
        ],
        out_specs=pl.BlockSpec((_SB, B, D), lambda i: (i, 0, 0)),
        out_shape=jax.ShapeDtypeStruct((S, B, D), jnp.float32),
    )(sums, gamma.reshape(1, D), beta.reshape(1, D), x, pos_table[:S])

    return out
